# Initial kernel scaffold; baseline (speedup 1.0000x reference)
import jax
import jax.numpy as jnp
from jax.experimental import pallas as pl


def kernel(x, state, batch_idxs, actor_ids, story_stop_idxs, W_ih, W_hh, b_ih, b_hh):
    raise NotImplementedError("write your pallas kernel here")


# speedup vs baseline: 1.7479x; 1.7479x over previous
import functools

import jax
import jax.numpy as jnp
from jax import lax
from jax.experimental import pallas as pl
from jax.experimental.pallas import tpu as pltpu
from jax.experimental.pallas import tpu_sc as plsc
from jax._src.pallas import mpmd as _mpmd

INPUT_SIZE = 64
HIDDEN = 64
CAST = 512
N_STORIES = 1024
M = N_STORIES * CAST
B = 16384
N_STOP = 64

NC = 2
NS = 16
NW = NC * NS
BPW = B // NW
CHUNK = 128
NCHUNK = BPW // CHUNK

_mesh = plsc.VectorSubcoreMesh(
    core_axis_name="c", subcore_axis_name="s", num_cores=NC, num_subcores=NS
)
_sc_params = pltpu.CompilerParams(use_tc_tiling_on_sc=False)


def _wid():
  return lax.axis_index("s") * NC + lax.axis_index("c")


def _sc_gather_body(state_hbm, idx_hbm, out_hbm, idx_v, rows_v, sem):
  wid = _wid()
  pltpu.sync_copy(idx_hbm.at[pl.ds(wid * NCHUNK, NCHUNK)], idx_v)
  descs = []
  for j in range(NCHUNK):
    descs.append(
        pltpu.async_copy(
            state_hbm.at[idx_v.at[j]],
            rows_v.at[pl.ds(j * CHUNK, CHUNK)],
            sem,
        )
    )
  for d in descs:
    d.wait()
  pltpu.sync_copy(rows_v, out_hbm.at[pl.ds(wid * BPW, BPW)])


_sc_gather = pl.kernel(
    _sc_gather_body,
    out_type=jax.ShapeDtypeStruct((B, HIDDEN), jnp.float32),
    mesh=_mesh,
    scratch_types=[
        pltpu.VMEM((NCHUNK, CHUNK), jnp.int32),
        pltpu.VMEM((BPW, HIDDEN), jnp.float32),
        pltpu.SemaphoreType.DMA,
    ],
    compiler_params=_sc_params,
)


def _sc_scatter_body(tbl_in, idx_hbm, rows_hbm, out_hbm, idx_v, rows_v, sem):
  del tbl_in
  wid = _wid()
  pltpu.sync_copy(idx_hbm.at[pl.ds(wid * NCHUNK, NCHUNK)], idx_v)
  pltpu.sync_copy(rows_hbm.at[pl.ds(wid * BPW, BPW)], rows_v)
  descs = []
  for j in range(NCHUNK):
    descs.append(
        pltpu.async_copy(
            rows_v.at[pl.ds(j * CHUNK, CHUNK)],
            out_hbm.at[idx_v.at[j]],
            sem,
        )
    )
  for d in descs:
    d.wait()


_sc_scatter = _mpmd._mpmd_map(
    [(_mesh, _sc_scatter_body)],
    out_types=jax.ShapeDtypeStruct((M, HIDDEN), jnp.float32),
    input_output_aliases={0: 0},
    scratch_types=[
        pltpu.VMEM((NCHUNK, CHUNK), jnp.int32),
        pltpu.VMEM((BPW, HIDDEN), jnp.float32),
        pltpu.SemaphoreType.DMA,
    ],
    compiler_params=_sc_params,
)


def _sc_zero_body(tbl_in, stop_hbm, zeros_hbm, out_hbm, stop_v, zeros_v):
  del tbl_in
  @pl.when(_wid() == 0)
  def _():
    pltpu.sync_copy(stop_hbm, stop_v)
    pltpu.sync_copy(zeros_hbm, zeros_v)
    pltpu.sync_copy(zeros_v, out_hbm.at[stop_v])


_sc_zero = _mpmd._mpmd_map(
    [(_mesh, _sc_zero_body)],
    out_types=jax.ShapeDtypeStruct((M, HIDDEN), jnp.float32),
    input_output_aliases={0: 0},
    scratch_types=[
        pltpu.VMEM((N_STOP,), jnp.int32),
        pltpu.VMEM((N_STOP, HIDDEN), jnp.float32),
    ],
    compiler_params=_sc_params,
)


_GRU_BS = 2048


def _gru_body(x_ref, h_ref, wih_ref, whh_ref, bih_ref, bhh_ref, out_ref):
  x = x_ref[...]
  h = h_ref[...]
  dn = (((1,), (1,)), ((), ()))
  gi = lax.dot_general(x, wih_ref[...], dn,
                       preferred_element_type=jnp.float32) + bih_ref[...]
  gh = lax.dot_general(h, whh_ref[...], dn,
                       preferred_element_type=jnp.float32) + bhh_ref[...]
  i_r, i_z, i_n = gi[:, :HIDDEN], gi[:, HIDDEN:2 * HIDDEN], gi[:, 2 * HIDDEN:]
  h_r, h_z, h_n = gh[:, :HIDDEN], gh[:, HIDDEN:2 * HIDDEN], gh[:, 2 * HIDDEN:]
  r = jax.nn.sigmoid(i_r + h_r)
  z = jax.nn.sigmoid(i_z + h_z)
  n = jnp.tanh(i_n + r * h_n)
  out_ref[...] = (1.0 - z) * n + z * h


_gru = pl.pallas_call(
    _gru_body,
    grid=(B // _GRU_BS,),
    in_specs=[
        pl.BlockSpec((_GRU_BS, INPUT_SIZE), lambda i: (i, 0)),
        pl.BlockSpec((_GRU_BS, HIDDEN), lambda i: (i, 0)),
        pl.BlockSpec((3 * HIDDEN, INPUT_SIZE), lambda i: (0, 0)),
        pl.BlockSpec((3 * HIDDEN, HIDDEN), lambda i: (0, 0)),
        pl.BlockSpec((1, 3 * HIDDEN), lambda i: (0, 0)),
        pl.BlockSpec((1, 3 * HIDDEN), lambda i: (0, 0)),
    ],
    out_specs=pl.BlockSpec((_GRU_BS, HIDDEN), lambda i: (i, 0)),
    out_shape=jax.ShapeDtypeStruct((B, HIDDEN), jnp.float32),
)


_COPY_BS = 8192


def _copy_body(s_ref, o_ref):
  o_ref[...] = s_ref[...]


_copy = pl.pallas_call(
    _copy_body,
    grid=(M // _COPY_BS,),
    in_specs=[pl.BlockSpec((_COPY_BS, HIDDEN), lambda i: (i, 0))],
    out_specs=pl.BlockSpec((_COPY_BS, HIDDEN), lambda i: (i, 0)),
    out_shape=jax.ShapeDtypeStruct((M, HIDDEN), jnp.float32),
)


def kernel(x, state, batch_idxs, actor_ids, story_stop_idxs, W_ih, W_hh,
           b_ih, b_hh):
  aid = jnp.clip(actor_ids, 0, CAST - 1).astype(jnp.int32)
  idxs = batch_idxs.astype(jnp.int32) * CAST + aid

  pos = jnp.arange(B, dtype=jnp.int32)
  winner = jnp.zeros((M,), jnp.int32).at[idxs].max(pos)
  keep = winner[idxs] == pos
  stop0 = story_stop_idxs[0].astype(jnp.int32)
  scat_idx = jnp.where(keep, idxs, stop0)

  selected = _sc_gather(state, idxs.reshape(B // CHUNK, CHUNK))
  new_selected = _gru(x, selected, W_ih, W_hh,
                      b_ih.reshape(1, 3 * HIDDEN), b_hh.reshape(1, 3 * HIDDEN))
  copied = _copy(state)
  scattered = _sc_scatter(copied, scat_idx.reshape(B // CHUNK, CHUNK),
                          new_selected)
  new_state = _sc_zero(scattered, story_stop_idxs.astype(jnp.int32),
                       jnp.zeros((N_STOP, HIDDEN), jnp.float32))
  return new_selected, new_state
